# baseline (speedup 1.0000x reference)
import functools

import jax
import jax.numpy as jnp
from jax import lax
from jax.experimental import pallas as pl
from jax.experimental.pallas import tpu as pltpu
from jax.experimental.pallas import tpu_sc as plsc

EMBED_DIM = 32
NUM_CORES = 2
NUM_SUBCORES = 16
NUM_WORKERS = NUM_CORES * NUM_SUBCORES
CHUNK = 256
ROW_BUFS = 4


def _make_kernel(n_total: int, vocab: int):
  per_w = n_total // NUM_WORKERS
  n_chunks = per_w // CHUNK
  mesh = plsc.VectorSubcoreMesh(core_axis_name="c", subcore_axis_name="s")

  @functools.partial(
      pl.kernel,
      mesh=mesh,
      out_type=jax.ShapeDtypeStruct((n_total, EMBED_DIM), jnp.float32),
      scratch_types=[
          pltpu.VMEM((per_w,), jnp.int32),
          pltpu.VMEM((ROW_BUFS * CHUNK, EMBED_DIM), jnp.float32),
          pltpu.SemaphoreType.DMA,
          pltpu.SemaphoreType.DMA,
      ],
      compiler_params=pltpu.CompilerParams(use_tc_tiling_on_sc=False),
  )
  def k(ids_hbm, remap_hbm, emb_hbm, out_hbm, idx_all, rows_v, sem_g2, sem_s):
    sid = lax.axis_index("s")
    wid = sid * NUM_CORES + lax.axis_index("c")
    base = wid * per_w
    pltpu.sync_copy(ids_hbm.at[pl.ds(base, per_w)], idx_all)

    def rows_sl(j):
      return rows_v.at[pl.ds(lax.rem(j, ROW_BUFS) * CHUNK, CHUNK)]

    def g2(j):
      idx_sl = idx_all.at[pl.ds(j * CHUNK, CHUNK)]
      return pltpu.make_async_copy(emb_hbm.at[idx_sl], rows_sl(j), sem_g2)

    def st(j):
      out_sl = out_hbm.at[pl.ds(base + j * CHUNK, CHUNK)]
      return pltpu.make_async_copy(rows_sl(j), out_sl, sem_s)

    g2(0).start()
    g2(1).start()
    g2(0).wait()
    st(0).start()
    g2(2).start()
    g2(1).wait()
    st(1).start()

    def body(j, carry):
      st(j - 2).wait()
      g2(j + 1).start()
      g2(j).wait()
      st(j).start()
      return carry

    lax.fori_loop(2, n_chunks - 1, body, 0)

    jl = n_chunks - 1
    st(jl - 2).wait()
    g2(jl).wait()
    st(jl).start()
    st(jl - 1).wait()
    st(jl).wait()

  return k


def kernel(client_ids, item_ids, item_id2graph_id, item_embeddings):
  del client_ids
  batch, seq_len = item_ids.shape
  n_total = batch * seq_len
  vocab = item_id2graph_id.shape[0]
  flat_ids = jnp.arange(n_total, dtype=jnp.int32) % vocab
  out = _make_kernel(n_total, vocab)(flat_ids, item_id2graph_id,
                                     item_embeddings)
  return out.reshape(batch, seq_len, EMBED_DIM)


# speedup vs baseline: 1.0015x; 1.0015x over previous
import functools

import jax
import jax.numpy as jnp
from jax import lax
from jax.experimental import pallas as pl
from jax.experimental.pallas import tpu as pltpu
from jax.experimental.pallas import tpu_sc as plsc

EMBED_DIM = 32
NUM_CORES = 2
NUM_SUBCORES = 16
NUM_WORKERS = NUM_CORES * NUM_SUBCORES
CHUNK = 320
IDS_BUFS = 6
GID_BUFS = 5
ROW_BUFS = 5


def _make_kernel(n_total: int, vocab: int):
  per_w = n_total // NUM_WORKERS
  n_chunks = per_w // CHUNK
  assert n_chunks >= 8 and per_w % CHUNK == 0 and CHUNK % 8 == 0
  stage_ch = (vocab // NUM_SUBCORES) // 8 * 8
  stage_last = vocab - (NUM_SUBCORES - 1) * stage_ch
  mesh = plsc.VectorSubcoreMesh(core_axis_name="c", subcore_axis_name="s")

  @functools.partial(
      pl.kernel,
      mesh=mesh,
      out_type=jax.ShapeDtypeStruct((n_total, EMBED_DIM), jnp.float32),
      scratch_types=[
          pltpu.VMEM_SHARED((vocab,), jnp.int32),
          pltpu.VMEM((IDS_BUFS * CHUNK,), jnp.int32),
          pltpu.VMEM((GID_BUFS * CHUNK,), jnp.int32),
          pltpu.VMEM((ROW_BUFS * CHUNK, EMBED_DIM), jnp.float32),
          pltpu.SemaphoreType.DMA,
          pltpu.SemaphoreType.DMA,
          pltpu.SemaphoreType.DMA,
          pltpu.SemaphoreType.DMA,
      ],
      compiler_params=pltpu.CompilerParams(use_tc_tiling_on_sc=False),
  )
  def k(ids_hbm, remap_hbm, emb_hbm, out_hbm, remap_sh, ids_v, gid_v,
        rows_v, sem_a, sem_g1, sem_g2, sem_s):
    sid = lax.axis_index("s")
    wid = sid * NUM_CORES + lax.axis_index("c")
    base = wid * per_w

    @pl.when(sid < NUM_SUBCORES - 1)
    def _():
      off = sid * stage_ch
      pltpu.sync_copy(remap_hbm.at[pl.ds(off, stage_ch)],
                      remap_sh.at[pl.ds(off, stage_ch)])

    @pl.when(sid == NUM_SUBCORES - 1)
    def _():
      off = (NUM_SUBCORES - 1) * stage_ch
      pltpu.sync_copy(remap_hbm.at[pl.ds(off, stage_last)],
                      remap_sh.at[pl.ds(off, stage_last)])

    plsc.subcore_barrier()

    def ids_sl(j):
      return ids_v.at[pl.ds(lax.rem(j, IDS_BUFS) * CHUNK, CHUNK)]

    def gid_sl(j):
      return gid_v.at[pl.ds(lax.rem(j, GID_BUFS) * CHUNK, CHUNK)]

    def rows_sl(j):
      return rows_v.at[pl.ds(lax.rem(j, ROW_BUFS) * CHUNK, CHUNK)]

    def a(j):
      ids_hbm_sl = ids_hbm.at[pl.ds(base + j * CHUNK, CHUNK)]
      return pltpu.make_async_copy(ids_hbm_sl, ids_sl(j), sem_a)

    def g1(j):
      return pltpu.make_async_copy(remap_sh.at[ids_sl(j)], gid_sl(j), sem_g1)

    def g2(j):
      return pltpu.make_async_copy(emb_hbm.at[gid_sl(j)], rows_sl(j), sem_g2)

    def st(j):
      out_sl = out_hbm.at[pl.ds(base + j * CHUNK, CHUNK)]
      return pltpu.make_async_copy(rows_sl(j), out_sl, sem_s)

    a(0).start()
    a(1).start()
    a(2).start()
    a(3).start()
    a(0).wait()
    g1(0).start()
    a(1).wait()
    g1(1).start()
    g1(0).wait()
    g2(0).start()
    g1(1).wait()
    g2(1).start()
    a(4).start()
    a(2).wait()
    g1(2).start()
    g1(2).wait()
    g2(2).start()

    a(5).start()
    a(3).wait()
    g1(3).start()
    g2(0).wait()
    st(0).start()
    g1(3).wait()
    g2(3).start()
    a(6).start()
    a(4).wait()
    g1(4).start()
    g2(1).wait()
    st(1).start()
    g1(4).wait()
    g2(4).start()

    def body(j, carry):
      a(j + 5).start()
      a(j + 3).wait()
      g1(j + 3).start()
      g2(j).wait()
      st(j).start()
      g1(j + 3).wait()
      st(j - 2).wait()
      g2(j + 3).start()
      return carry

    lax.fori_loop(2, n_chunks - 5, body, 0)

    m = n_chunks - 5
    a(m + 3).wait()
    g1(m + 3).start()
    g2(m).wait()
    st(m).start()
    g1(m + 3).wait()
    st(m - 2).wait()
    g2(m + 3).start()
    a(m + 4).wait()
    g1(m + 4).start()
    g2(m + 1).wait()
    st(m + 1).start()
    g1(m + 4).wait()
    st(m - 1).wait()
    g2(m + 4).start()
    g2(m + 2).wait()
    st(m + 2).start()
    g2(m + 3).wait()
    st(m + 3).start()
    g2(m + 4).wait()
    st(m + 4).start()
    st(m).wait()
    st(m + 1).wait()
    st(m + 2).wait()
    st(m + 3).wait()
    st(m + 4).wait()

  return k


def kernel(client_ids, item_ids, item_id2graph_id, item_embeddings):
  del client_ids
  batch, seq_len = item_ids.shape
  n_total = batch * seq_len
  vocab = item_id2graph_id.shape[0]
  flat_ids = item_ids.reshape(n_total)
  out = _make_kernel(n_total, vocab)(flat_ids, item_id2graph_id,
                                     item_embeddings)
  return out.reshape(batch, seq_len, EMBED_DIM)
